# baseline (speedup 1.0000x reference)
import functools

import jax
import jax.numpy as jnp
from jax import lax
from jax.experimental import pallas as pl
from jax.experimental.pallas import tpu as pltpu
from jax.experimental.pallas import tpu_sc as plsc

_T = 128
_FC = 512


def _sc_row_scatter(rows, dst, n_out):
    r, d = rows.shape
    info = plsc.get_sparse_core_info()
    nc, ns = info.num_cores, info.num_subcores
    nw = nc * ns
    b_per_w = r // nw
    mesh = plsc.VectorSubcoreMesh(core_axis_name="c", subcore_axis_name="s")

    @functools.partial(
        pl.kernel,
        mesh=mesh,
        out_type=jax.ShapeDtypeStruct((n_out, d), jnp.float32),
        scratch_types=[
            pltpu.VMEM((b_per_w,), jnp.int32),
            pltpu.VMEM((b_per_w, d), jnp.float32),
            pltpu.SemaphoreType.DMA,
        ],
    )
    def scatter(rows_hbm, dst_hbm, out_hbm, idx_v, rows_v, sem):
        wid = lax.axis_index("s") * nc + lax.axis_index("c")
        base = wid * b_per_w
        pltpu.sync_copy(dst_hbm.at[pl.ds(base, b_per_w)], idx_v)
        pltpu.sync_copy(rows_hbm.at[pl.ds(base, b_per_w)], rows_v)
        pltpu.async_copy(rows_v, out_hbm.at[idx_v], sem).wait()

    return scatter(rows, dst)


def _sc_row_gather(table, idx, n_out):
    rows, d = table.shape
    info = plsc.get_sparse_core_info()
    nc, ns = info.num_cores, info.num_subcores
    nw = nc * ns
    b_per_w = n_out // nw
    mesh = plsc.VectorSubcoreMesh(core_axis_name="c", subcore_axis_name="s")

    @functools.partial(
        pl.kernel,
        mesh=mesh,
        out_type=jax.ShapeDtypeStruct((n_out, d), jnp.float32),
        scratch_types=[
            pltpu.VMEM((b_per_w,), jnp.int32),
            pltpu.VMEM((b_per_w, d), jnp.float32),
            pltpu.SemaphoreType.DMA,
        ],
    )
    def gather(table_hbm, idx_hbm, out_hbm, idx_v, rows_v, sem):
        wid = lax.axis_index("s") * nc + lax.axis_index("c")
        base = wid * b_per_w
        pltpu.sync_copy(idx_hbm.at[pl.ds(base, b_per_w)], idx_v)
        pltpu.async_copy(table_hbm.at[idx_v], rows_v, sem).wait()
        pltpu.sync_copy(rows_v, out_hbm.at[pl.ds(base, b_per_w)])

    return gather(table, idx)


def _mlp_body(ts_ref, nt_ref, x_ref, w1_ref, b1_ref, w2_ref, b2_ref, out_ref,
              w1b_ref, w2b_ref):
    e = pl.program_id(0)
    f = pl.program_id(1)
    w1b_ref[...] = w1_ref[0].astype(jnp.bfloat16)
    w2b_ref[...] = w2_ref[0].astype(jnp.bfloat16)
    b1v = b1_ref[0, 0]
    b2v = b2_ref[0, 0]
    t0 = ts_ref[e]
    ntile = nt_ref[e]

    def tile_out(i):
        base = (t0 + i) * _T
        x = x_ref[pl.ds(base, _T), :].astype(jnp.bfloat16)
        h = jnp.dot(x, w1b_ref[...], preferred_element_type=jnp.float32)
        h = jnp.maximum(h + b1v, 0.0).astype(jnp.bfloat16)
        return base, jnp.dot(h, w2b_ref[...], preferred_element_type=jnp.float32)

    @pl.when(f == 0)
    def _():
        def body(i, _):
            base, y = tile_out(i)
            out_ref[pl.ds(base, _T), :] = y + b2v
            return 0
        lax.fori_loop(0, ntile, body, 0)

    @pl.when(f != 0)
    def _():
        def body(i, _):
            base, y = tile_out(i)
            out_ref[pl.ds(base, _T), :] += y
            return 0
        lax.fori_loop(0, ntile, body, 0)


def _grouped_mlp(x_sorted, tile_start, tile_count, W1, b1, W2, b2):
    p, d = x_sorted.shape
    e, _, f_dim = W1.shape
    nf = f_dim // _FC
    b1r = b1.reshape(e, 1, f_dim)
    b2r = b2.reshape(e, 1, d)

    grid_spec = pltpu.PrefetchScalarGridSpec(
        num_scalar_prefetch=2,
        grid=(e, nf),
        in_specs=[
            pl.BlockSpec((p, d), lambda ei, f, ts, nt: (0, 0)),
            pl.BlockSpec((1, d, _FC), lambda ei, f, ts, nt: (ei, 0, f)),
            pl.BlockSpec((1, 1, _FC), lambda ei, f, ts, nt: (ei, 0, f)),
            pl.BlockSpec((1, _FC, d), lambda ei, f, ts, nt: (ei, f, 0)),
            pl.BlockSpec((1, 1, d), lambda ei, f, ts, nt: (ei, 0, 0)),
        ],
        out_specs=pl.BlockSpec((p, d), lambda ei, f, ts, nt: (0, 0)),
        scratch_shapes=[
            pltpu.VMEM((d, _FC), jnp.bfloat16),
            pltpu.VMEM((_FC, d), jnp.bfloat16),
        ],
    )
    return pl.pallas_call(
        _mlp_body,
        grid_spec=grid_spec,
        out_shape=jax.ShapeDtypeStruct((p, d), jnp.float32),
    )(tile_start, tile_count, x_sorted, W1, b1r, W2, b2r)


def _route(flat_indices, n_experts, p_total):
    s = flat_indices.shape[0]
    nt = p_total // _T
    idx = flat_indices.astype(jnp.int32)
    onehot = (idx[:, None] == jnp.arange(n_experts, dtype=jnp.int32)[None, :])
    blk = 128
    nb = s // blk
    ohb = onehot.astype(jnp.float32).reshape(nb, blk, n_experts)
    ltri = (
        jnp.arange(blk, dtype=jnp.int32)[:, None]
        >= jnp.arange(blk, dtype=jnp.int32)[None, :]
    ).astype(jnp.float32)
    c_in = jax.lax.dot_general(
        ltri, ohb, (((1,), (1,)), ((), ())),
        preferred_element_type=jnp.float32,
    )
    c_in = jnp.moveaxis(c_in, 0, 1)
    tot = c_in[:, -1, :]
    blk_off = jnp.cumsum(tot, axis=0) - tot
    csum = (c_in + blk_off[:, None, :]).reshape(s, n_experts).astype(jnp.int32)
    counts = csum[-1]
    pad_counts = ((counts + _T - 1) // _T) * _T
    zero = jnp.zeros((1,), jnp.int32)
    pad_off = jnp.concatenate([zero, jnp.cumsum(pad_counts).astype(jnp.int32)])
    pos = jnp.sum(
        jnp.where(onehot, csum + pad_off[None, :-1], 0), axis=1
    ).astype(jnp.int32) - 1
    tile_start = (pad_off[:-1] // _T).astype(jnp.int32)
    tile_count = (pad_counts // _T).astype(jnp.int32)
    return pos, tile_start, tile_count


def kernel(hidden_states, expert_indices, W1, b1, W2, b2):
    bsz, seq, d = hidden_states.shape
    e = W1.shape[0]
    s = bsz * seq
    p_total = ((s + e * (_T - 1) + 255) // 256) * 256
    p_total = ((p_total + _T - 1) // _T) * _T

    flat = hidden_states.reshape(s, d)
    pos, tile_start, tile_count = _route(
        expert_indices.reshape(-1), e, p_total
    )
    x_sorted = _sc_row_scatter(flat, pos, p_total)
    y_sorted = _grouped_mlp(x_sorted, tile_start, tile_count, W1, b1, W2, b2)
    out = _sc_row_gather(y_sorted, pos, s)
    return out.reshape(bsz, seq, d)


# speedup vs baseline: 1.2174x; 1.2174x over previous
import functools

import jax
import jax.numpy as jnp
from jax import lax
from jax.experimental import pallas as pl
from jax.experimental.pallas import tpu as pltpu
from jax.experimental.pallas import tpu_sc as plsc

_T = 128
_FC = 1024


def _sc_row_scatter(rows, dst, n_out):
    r, d = rows.shape
    info = plsc.get_sparse_core_info()
    nc, ns = info.num_cores, info.num_subcores
    nw = nc * ns
    b_per_w = r // nw
    mesh = plsc.VectorSubcoreMesh(core_axis_name="c", subcore_axis_name="s")

    @functools.partial(
        pl.kernel,
        mesh=mesh,
        out_type=jax.ShapeDtypeStruct((n_out, d), jnp.float32),
        scratch_types=[
            pltpu.VMEM((b_per_w,), jnp.int32),
            pltpu.VMEM((b_per_w, d), jnp.float32),
            pltpu.SemaphoreType.DMA,
        ],
    )
    def scatter(rows_hbm, dst_hbm, out_hbm, idx_v, rows_v, sem):
        wid = lax.axis_index("s") * nc + lax.axis_index("c")
        base = wid * b_per_w
        pltpu.sync_copy(dst_hbm.at[pl.ds(base, b_per_w)], idx_v)
        pltpu.sync_copy(rows_hbm.at[pl.ds(base, b_per_w)], rows_v)
        pltpu.async_copy(rows_v, out_hbm.at[idx_v], sem).wait()

    return scatter(rows, dst)


def _sc_row_gather(table, idx, n_out):
    rows, d = table.shape
    info = plsc.get_sparse_core_info()
    nc, ns = info.num_cores, info.num_subcores
    nw = nc * ns
    b_per_w = n_out // nw
    mesh = plsc.VectorSubcoreMesh(core_axis_name="c", subcore_axis_name="s")

    @functools.partial(
        pl.kernel,
        mesh=mesh,
        out_type=jax.ShapeDtypeStruct((n_out, d), jnp.float32),
        scratch_types=[
            pltpu.VMEM((b_per_w,), jnp.int32),
            pltpu.VMEM((b_per_w, d), jnp.float32),
            pltpu.SemaphoreType.DMA,
        ],
    )
    def gather(table_hbm, idx_hbm, out_hbm, idx_v, rows_v, sem):
        wid = lax.axis_index("s") * nc + lax.axis_index("c")
        base = wid * b_per_w
        pltpu.sync_copy(idx_hbm.at[pl.ds(base, b_per_w)], idx_v)
        pltpu.async_copy(table_hbm.at[idx_v], rows_v, sem).wait()
        pltpu.sync_copy(rows_v, out_hbm.at[pl.ds(base, b_per_w)])

    return gather(table, idx)


def _mlp_body(ts_ref, nt_ref, x_ref, w1_ref, b1_ref, w2_ref, b2_ref, out_ref,
              w1b_ref, w2b_ref):
    e = pl.program_id(0)
    f = pl.program_id(1)
    w1b_ref[...] = w1_ref[0].astype(jnp.bfloat16)
    w2b_ref[...] = w2_ref[0].astype(jnp.bfloat16)
    b1v = b1_ref[0, 0]
    b2v = b2_ref[0, 0]
    t0 = ts_ref[e]
    ntile = nt_ref[e]

    def tile_out(i):
        base = (t0 + i) * _T
        x = x_ref[pl.ds(base, _T), :].astype(jnp.bfloat16)
        h = jnp.dot(x, w1b_ref[...], preferred_element_type=jnp.float32)
        h = jnp.maximum(h + b1v, 0.0).astype(jnp.bfloat16)
        return base, jnp.dot(h, w2b_ref[...], preferred_element_type=jnp.float32)

    @pl.when(f == 0)
    def _():
        def body(i, _):
            base, y = tile_out(i)
            out_ref[pl.ds(base, _T), :] = y + b2v
            return 0
        lax.fori_loop(0, ntile, body, 0)

    @pl.when(f != 0)
    def _():
        def body(i, _):
            base, y = tile_out(i)
            out_ref[pl.ds(base, _T), :] += y
            return 0
        lax.fori_loop(0, ntile, body, 0)


def _grouped_mlp(x_sorted, tile_start, tile_count, W1, b1, W2, b2):
    p, d = x_sorted.shape
    e, _, f_dim = W1.shape
    nf = f_dim // _FC
    b1r = b1.reshape(e, 1, f_dim)
    b2r = b2.reshape(e, 1, d)

    grid_spec = pltpu.PrefetchScalarGridSpec(
        num_scalar_prefetch=2,
        grid=(e, nf),
        in_specs=[
            pl.BlockSpec((p, d), lambda ei, f, ts, nt: (0, 0)),
            pl.BlockSpec((1, d, _FC), lambda ei, f, ts, nt: (ei, 0, f)),
            pl.BlockSpec((1, 1, _FC), lambda ei, f, ts, nt: (ei, 0, f)),
            pl.BlockSpec((1, _FC, d), lambda ei, f, ts, nt: (ei, f, 0)),
            pl.BlockSpec((1, 1, d), lambda ei, f, ts, nt: (ei, 0, 0)),
        ],
        out_specs=pl.BlockSpec((p, d), lambda ei, f, ts, nt: (0, 0)),
        scratch_shapes=[
            pltpu.VMEM((d, _FC), jnp.bfloat16),
            pltpu.VMEM((_FC, d), jnp.bfloat16),
        ],
    )
    return pl.pallas_call(
        _mlp_body,
        grid_spec=grid_spec,
        out_shape=jax.ShapeDtypeStruct((p, d), jnp.float32),
    )(tile_start, tile_count, x_sorted, W1, b1r, W2, b2r)


def _route(flat_indices, n_experts, p_total):
    s = flat_indices.shape[0]
    idx = flat_indices.astype(jnp.int32)
    onehot = (idx[:, None] == jnp.arange(n_experts, dtype=jnp.int32)[None, :])
    blk = 128
    nb = s // blk
    ohb = onehot.astype(jnp.float32).reshape(nb, blk, n_experts)
    ltri = (
        jnp.arange(blk, dtype=jnp.int32)[:, None]
        >= jnp.arange(blk, dtype=jnp.int32)[None, :]
    ).astype(jnp.float32)
    c_in = jax.lax.dot_general(
        ltri, ohb, (((1,), (1,)), ((), ())),
        preferred_element_type=jnp.float32,
    )
    c_in = jnp.moveaxis(c_in, 0, 1)
    tot = c_in[:, -1, :]
    blk_off = jnp.cumsum(tot, axis=0) - tot
    csum = (c_in + blk_off[:, None, :]).reshape(s, n_experts).astype(jnp.int32)
    counts = csum[-1]
    pad_counts = ((counts + _T - 1) // _T) * _T
    zero = jnp.zeros((1,), jnp.int32)
    pad_off = jnp.concatenate([zero, jnp.cumsum(pad_counts).astype(jnp.int32)])
    pos = jnp.sum(
        jnp.where(onehot, csum + pad_off[None, :-1], 0), axis=1
    ).astype(jnp.int32) - 1
    tile_start = (pad_off[:-1] // _T).astype(jnp.int32)
    tile_count = (pad_counts // _T).astype(jnp.int32)
    return pos, tile_start, tile_count


def kernel(hidden_states, expert_indices, W1, b1, W2, b2):
    bsz, seq, d = hidden_states.shape
    e = W1.shape[0]
    s = bsz * seq
    p_total = ((s + e * (_T - 1) + 255) // 256) * 256
    p_total = ((p_total + _T - 1) // _T) * _T

    flat = hidden_states.reshape(s, d)
    pos, tile_start, tile_count = _route(
        expert_indices.reshape(-1), e, p_total
    )
    x_sorted = _sc_row_scatter(flat, pos, p_total)
    y_sorted = _grouped_mlp(x_sorted, tile_start, tile_count, W1, b1, W2, b2)
    out = _sc_row_gather(y_sorted, pos, s)
    return out.reshape(bsz, seq, d)
